# baseline (speedup 1.0000x reference)
import jax
import jax.numpy as jnp
from jax.experimental import pallas as pl
from jax.experimental.pallas import tpu as pltpu

CHANNELS = 768
N_CLUSTERS = 512
N_TOKENS = 32768
BT = 2048
N_BLOCKS = N_TOKENS // BT
EXT = CHANNELS + 128


def _cluster_body(x_ref, w_ref, b_ref, out_ref, acc_ref):
    i = pl.program_id(0)

    @pl.when(i == 0)
    def _init():
        acc_ref[...] = jnp.zeros_like(acc_ref)

    xb = x_ref[...]
    logits = (
        jnp.dot(xb, w_ref[...].T, preferred_element_type=jnp.float32)
        + b_ref[...]
    )
    rowmax = jnp.max(logits, axis=1, keepdims=True)
    onehot = (logits == rowmax).astype(jnp.bfloat16)
    xb_ext = jnp.concatenate(
        [xb.astype(jnp.bfloat16), jnp.ones((BT, 128), jnp.bfloat16)], axis=1
    )
    acc_ref[...] += jax.lax.dot_general(
        onehot,
        xb_ext,
        (((0,), (0,)), ((), ())),
        preferred_element_type=jnp.float32,
    )

    @pl.when(i == N_BLOCKS - 1)
    def _finalize():
        out_ref[...] = acc_ref[:, :CHANNELS] / acc_ref[:, CHANNELS:CHANNELS + 1]


@jax.jit
def kernel(x, W, b):
    out = pl.pallas_call(
        _cluster_body,
        grid=(N_BLOCKS,),
        in_specs=[
            pl.BlockSpec((BT, CHANNELS), lambda i: (i, 0)),
            pl.BlockSpec((N_CLUSTERS, CHANNELS), lambda i: (0, 0)),
            pl.BlockSpec((1, N_CLUSTERS), lambda i: (0, 0)),
        ],
        out_specs=pl.BlockSpec((N_CLUSTERS, CHANNELS), lambda i: (0, 0)),
        out_shape=jax.ShapeDtypeStruct((N_CLUSTERS, CHANNELS), jnp.float32),
        scratch_shapes=[pltpu.VMEM((N_CLUSTERS, EXT), jnp.float32)],
    )(x, W, b.reshape(1, N_CLUSTERS))
    return out


# speedup vs baseline: 1.0135x; 1.0135x over previous
import jax
import jax.numpy as jnp
from jax.experimental import pallas as pl
from jax.experimental.pallas import tpu as pltpu

CHANNELS = 768
N_CLUSTERS = 512
N_TOKENS = 32768
BT = 4096
N_BLOCKS = N_TOKENS // BT
EXT = CHANNELS + 128


def _cluster_body(x_ref, w_ref, b_ref, out_ref, acc_ref):
    i = pl.program_id(0)

    @pl.when(i == 0)
    def _init():
        acc_ref[...] = jnp.zeros_like(acc_ref)

    xb = x_ref[...]
    logits = (
        jnp.dot(xb, w_ref[...].T, preferred_element_type=jnp.float32)
        + b_ref[...]
    )
    rowmax = jnp.max(logits, axis=1, keepdims=True)
    onehot = (logits == rowmax).astype(jnp.bfloat16)
    xb_ext = jnp.concatenate(
        [xb.astype(jnp.bfloat16), jnp.ones((BT, 128), jnp.bfloat16)], axis=1
    )
    acc_ref[...] += jax.lax.dot_general(
        onehot,
        xb_ext,
        (((0,), (0,)), ((), ())),
        preferred_element_type=jnp.float32,
    )

    @pl.when(i == N_BLOCKS - 1)
    def _finalize():
        out_ref[...] = acc_ref[:, :CHANNELS] / acc_ref[:, CHANNELS:CHANNELS + 1]


@jax.jit
def kernel(x, W, b):
    out = pl.pallas_call(
        _cluster_body,
        grid=(N_BLOCKS,),
        in_specs=[
            pl.BlockSpec((BT, CHANNELS), lambda i: (i, 0)),
            pl.BlockSpec((N_CLUSTERS, CHANNELS), lambda i: (0, 0)),
            pl.BlockSpec((1, N_CLUSTERS), lambda i: (0, 0)),
        ],
        out_specs=pl.BlockSpec((N_CLUSTERS, CHANNELS), lambda i: (0, 0)),
        out_shape=jax.ShapeDtypeStruct((N_CLUSTERS, CHANNELS), jnp.float32),
        scratch_shapes=[pltpu.VMEM((N_CLUSTERS, EXT), jnp.float32)],
    )(x, W, b.reshape(1, N_CLUSTERS))
    return out
